# baseline (speedup 1.0000x reference)
import jax
import jax.numpy as jnp
from jax.experimental import pallas as pl
from jax.experimental.pallas import tpu as pltpu

_K = 16
_EPS = 1.1754943508222875e-38
_L = 128


def _recur_body(x_ref, khot_ref, w_ref):
    r, n = x_ref.shape
    m = jnp.max(x_ref[...], axis=-1, keepdims=True)
    w0 = jnp.exp(x_ref[...] - m)
    w_ref[...] = w0
    s = jnp.sum(w0, axis=-1, keepdims=True)
    eps = jnp.float32(_EPS)
    invs = []
    for t in range(_K):
        inv_s = 1.0 / s
        invs.append(inv_s)
        if t < _K - 1:
            w = w_ref[...]
            wn = w * jnp.maximum(1.0 - w * inv_s, eps)
            w_ref[...] = wn
            s = jnp.sum(wn, axis=-1, keepdims=True)
    strip = 2048
    for a in range(0, n, strip):
        b_ = min(n, a + strip)
        xs = x_ref[:, a:b_]
        ws = jnp.exp(xs - m)
        kh = jnp.zeros_like(ws)
        for t in range(_K):
            p = ws * invs[t]
            kh = kh + p
            if t < _K - 1:
                ws = ws * jnp.maximum(1.0 - p, eps)
        khot_ref[:, a:b_] = kh


def _select_body(khot_ref, pert_ref, comp_ref, gidx_ref, hard_ref,
                 vals_ref, mc_ref):
    r, n = khot_ref.shape
    nchunks = mc_ref.shape[1]
    npad = hard_ref.shape[1]
    neg_inf = jnp.float32(-jnp.inf)

    hard_ref[...] = jnp.zeros((r, npad), jnp.float32)
    vals_ref[:, :n] = khot_ref[...]
    if npad > n:
        vals_ref[:, n:] = jnp.zeros((r, npad - n), jnp.float32)

    for c in range(nchunks):
        mc_ref[:, c:c + 1] = jnp.max(vals_ref[:, c * _L:(c + 1) * _L],
                                     axis=-1, keepdims=True)

    mchunk = mc_ref[...]
    ic = jax.lax.broadcasted_iota(jnp.int32, (r, nchunks), 1)
    chunk_firsts = []
    for t in range(_K):
        cmx = jnp.max(mchunk, axis=-1, keepdims=True)
        cand = jnp.where(mchunk == cmx, ic, jnp.int32(nchunks))
        firstc = jnp.min(cand, axis=-1, keepdims=True)
        chunk_firsts.append(firstc)
        mchunk = jnp.where(ic == firstc, neg_inf, mchunk)

    lane = jax.lax.iota(jnp.int32, _L)
    for t in range(_K):
        fc = chunk_firsts[t]
        for row in range(r):
            c = jnp.min(fc[row:row + 1, :])
            base = pl.multiple_of(c * _L, _L)
            comp_ref[row, t * _L:(t + 1) * _L] = vals_ref[row, pl.ds(base, _L)]
            gidx_ref[row, t * _L:(t + 1) * _L] = base + lane

    big = jnp.int32(2 ** 30)
    winners = []
    for t in range(_K):
        comp = comp_ref[...]
        gidx = gidx_ref[...]
        mx = jnp.max(comp, axis=-1, keepdims=True)
        cand = jnp.where(comp == mx, gidx, big)
        fg = jnp.min(cand, axis=-1, keepdims=True)
        winners.append((fg, mx))
        comp_ref[...] = jnp.where(gidx == fg, neg_inf, comp)

    for t in range(_K):
        fg, mx = winners[t]
        for row in range(r):
            g = jnp.min(fg[row:row + 1, :])
            base = pl.multiple_of(
                jax.lax.shift_left(jax.lax.shift_right_logical(g, 7), 7), _L)
            pos = g - base
            kv = jnp.min(mx[row:row + 1, :])
            val = (jnp.float32(1.0) - kv) + kv
            chunk = hard_ref[row, pl.ds(base, _L)]
            hard_ref[row, pl.ds(base, _L)] = jnp.where(lane == pos, val, chunk)

    pert_ref[...] = hard_ref[:, :n]


def kernel(logits):
    b, n = logits.shape
    rows = 8
    nchunks = (n + _L - 1) // _L
    npad = nchunks * _L
    f32 = jnp.float32
    khot = pl.pallas_call(
        _recur_body,
        grid=(b // rows,),
        in_specs=[pl.BlockSpec((rows, n), lambda i: (i, 0))],
        out_specs=pl.BlockSpec((rows, n), lambda i: (i, 0)),
        out_shape=jax.ShapeDtypeStruct((b, n), f32),
        scratch_shapes=[pltpu.VMEM((rows, n), f32)],
    )(logits)
    pert = pl.pallas_call(
        _select_body,
        grid=(b // rows,),
        in_specs=[pl.BlockSpec((rows, n), lambda i: (i, 0))],
        out_specs=pl.BlockSpec((rows, n), lambda i: (i, 0)),
        out_shape=jax.ShapeDtypeStruct((b, n), f32),
        scratch_shapes=[
            pltpu.VMEM((rows, _K * _L), f32),
            pltpu.VMEM((rows, _K * _L), jnp.int32),
            pltpu.VMEM((rows, npad), f32),
            pltpu.VMEM((rows, npad), f32),
            pltpu.VMEM((rows, nchunks), f32),
        ],
    )(khot)
    return pert, khot


# speedup vs baseline: 1.2267x; 1.2267x over previous
import jax
import jax.numpy as jnp
from jax.experimental import pallas as pl
from jax.experimental.pallas import tpu as pltpu

_K = 16
_EPS = 1.1754943508222875e-38
_L = 128


def _recur_body(x_ref, khot_ref, w_ref):
    r, n = x_ref.shape
    m = jnp.max(x_ref[...], axis=-1, keepdims=True)
    w0 = jnp.exp(x_ref[...] - m)
    w_ref[...] = w0
    khot_ref[...] = jnp.zeros((r, n), jnp.float32)
    s = jnp.sum(w0, axis=-1, keepdims=True)
    q = jnp.sum(w0 * w0, axis=-1, keepdims=True)
    eps = jnp.float32(_EPS)
    for j in range(_K // 2):
        r0 = 1.0 / s
        s1 = s - q * r0
        r1 = 1.0 / s1
        w = w_ref[...]
        p0 = w * r0
        w1 = w * jnp.maximum(1.0 - p0, eps)
        p1 = w1 * r1
        khot_ref[...] += p0 + p1
        if j < _K // 2 - 1:
            w2 = w1 * jnp.maximum(1.0 - p1, eps)
            w_ref[...] = w2
            s = jnp.sum(w2, axis=-1, keepdims=True)
            q = jnp.sum(w2 * w2, axis=-1, keepdims=True)


def _select_body(khot_ref, pert_ref, comp_ref, gidx_ref, hard_ref,
                 vals_ref, mc_ref):
    r, n = khot_ref.shape
    nchunks = mc_ref.shape[1]
    npad = hard_ref.shape[1]
    neg_inf = jnp.float32(-jnp.inf)

    hard_ref[...] = jnp.zeros((r, npad), jnp.float32)
    vals_ref[:, :n] = khot_ref[...]
    if npad > n:
        vals_ref[:, n:] = jnp.zeros((r, npad - n), jnp.float32)

    for c in range(nchunks):
        mc_ref[:, c:c + 1] = jnp.max(vals_ref[:, c * _L:(c + 1) * _L],
                                     axis=-1, keepdims=True)

    mchunk = mc_ref[...]
    ic = jax.lax.broadcasted_iota(jnp.int32, (r, nchunks), 1)
    chunk_firsts = []
    for t in range(_K):
        cmx = jnp.max(mchunk, axis=-1, keepdims=True)
        cand = jnp.where(mchunk == cmx, ic, jnp.int32(nchunks))
        firstc = jnp.min(cand, axis=-1, keepdims=True)
        chunk_firsts.append(firstc)
        mchunk = jnp.where(ic == firstc, neg_inf, mchunk)

    lane = jax.lax.iota(jnp.int32, _L)
    for t in range(_K):
        fc = chunk_firsts[t]
        for row in range(r):
            c = jnp.min(fc[row:row + 1, :])
            base = pl.multiple_of(c * _L, _L)
            comp_ref[row, t * _L:(t + 1) * _L] = vals_ref[row, pl.ds(base, _L)]
            gidx_ref[row, t * _L:(t + 1) * _L] = base + lane

    big = jnp.int32(2 ** 30)
    winners = []
    for t in range(_K):
        comp = comp_ref[...]
        gidx = gidx_ref[...]
        mx = jnp.max(comp, axis=-1, keepdims=True)
        cand = jnp.where(comp == mx, gidx, big)
        fg = jnp.min(cand, axis=-1, keepdims=True)
        winners.append((fg, mx))
        comp_ref[...] = jnp.where(gidx == fg, neg_inf, comp)

    for t in range(_K):
        fg, mx = winners[t]
        for row in range(r):
            g = jnp.min(fg[row:row + 1, :])
            base = pl.multiple_of(
                jax.lax.shift_left(jax.lax.shift_right_logical(g, 7), 7), _L)
            pos = g - base
            kv = jnp.min(mx[row:row + 1, :])
            val = (jnp.float32(1.0) - kv) + kv
            chunk = hard_ref[row, pl.ds(base, _L)]
            hard_ref[row, pl.ds(base, _L)] = jnp.where(lane == pos, val, chunk)

    pert_ref[...] = hard_ref[:, :n]


def kernel(logits):
    b, n = logits.shape
    rows = 8
    nchunks = (n + _L - 1) // _L
    npad = nchunks * _L
    f32 = jnp.float32
    khot = pl.pallas_call(
        _recur_body,
        grid=(b // rows,),
        in_specs=[pl.BlockSpec((rows, n), lambda i: (i, 0))],
        out_specs=pl.BlockSpec((rows, n), lambda i: (i, 0)),
        out_shape=jax.ShapeDtypeStruct((b, n), f32),
        scratch_shapes=[pltpu.VMEM((rows, n), f32)],
    )(logits)
    pert = pl.pallas_call(
        _select_body,
        grid=(b // rows,),
        in_specs=[pl.BlockSpec((rows, n), lambda i: (i, 0))],
        out_specs=pl.BlockSpec((rows, n), lambda i: (i, 0)),
        out_shape=jax.ShapeDtypeStruct((b, n), f32),
        scratch_shapes=[
            pltpu.VMEM((rows, _K * _L), f32),
            pltpu.VMEM((rows, _K * _L), jnp.int32),
            pltpu.VMEM((rows, npad), f32),
            pltpu.VMEM((rows, npad), f32),
            pltpu.VMEM((rows, nchunks), f32),
        ],
    )(khot)
    return pert, khot
